# Initial kernel scaffold; baseline (speedup 1.0000x reference)
import jax
import jax.numpy as jnp
from jax.experimental import pallas as pl


def kernel(x_cat_ids, tables):
    raise NotImplementedError("write your pallas kernel here")


# speedup vs baseline: 1.2115x; 1.2115x over previous
import functools

import jax
import jax.numpy as jnp
from jax import lax
from jax.experimental import pallas as pl
from jax.experimental.pallas import tpu as pltpu
from jax.experimental.pallas import tpu_sc as plsc

NUM_FIELDS = 26
VOCAB = 100000
EMB_DIM = 32
BATCH = 16384

ROWS = BATCH * NUM_FIELDS
NUM_WORKERS = 32
RPW = ROWS // NUM_WORKERS
CHUNK = 1664
NCHUNK = RPW // CHUNK
PERIOD = 208


def _sc_gather(ids_flat, offs, tables_flat):
    mesh = plsc.VectorSubcoreMesh(core_axis_name="c", subcore_axis_name="s")

    @functools.partial(
        pl.kernel,
        mesh=mesh,
        compiler_params=pltpu.CompilerParams(use_tc_tiling_on_sc=False),
        out_type=jax.ShapeDtypeStruct((ROWS, EMB_DIM), jnp.float32),
        scratch_types=[
            pltpu.VMEM((RPW,), jnp.int32),
            pltpu.VMEM((PERIOD,), jnp.int32),
            pltpu.VMEM((CHUNK, EMB_DIM), jnp.float32),
            pltpu.VMEM((CHUNK, EMB_DIM), jnp.float32),
            pltpu.SemaphoreType.DMA,
            pltpu.SemaphoreType.DMA,
        ],
    )
    def k(ids_hbm, offs_hbm, tab_hbm, out_hbm, idx_v, offs_v, buf0, buf1,
          sem0, sem1):
        wid = lax.axis_index("s") * 2 + lax.axis_index("c")
        base = wid * RPW
        pltpu.sync_copy(ids_hbm.at[pl.ds(base, RPW)], idx_v)
        pltpu.sync_copy(offs_hbm, offs_v)

        def add_offsets(o, carry):
            for kk in range(PERIOD // 16):
                sl = pl.ds(o * PERIOD + kk * 16, 16)
                idx_v[sl] = idx_v[sl] + offs_v[pl.ds(kk * 16, 16)]
            return carry

        lax.fori_loop(0, RPW // PERIOD, add_offsets, 0)

        bufs = (buf0, buf1)
        sems = (sem0, sem1)
        copies = [None] * NCHUNK
        copies[0] = pltpu.async_copy(
            tab_hbm.at[idx_v.at[pl.ds(0, CHUNK)]], bufs[0], sems[0])
        for c in range(NCHUNK):
            copies[c].wait()
            if c + 1 < NCHUNK:
                nb = (c + 1) % 2
                copies[c + 1] = pltpu.async_copy(
                    tab_hbm.at[idx_v.at[pl.ds((c + 1) * CHUNK, CHUNK)]],
                    bufs[nb], sems[nb])
            pltpu.sync_copy(bufs[c % 2],
                            out_hbm.at[pl.ds(base + c * CHUNK, CHUNK)])

    return k(ids_flat, offs, tables_flat)


def kernel(x_cat_ids, tables):
    ids_flat = x_cat_ids.astype(jnp.int32).reshape(-1)
    offs = jnp.tile(jnp.arange(NUM_FIELDS, dtype=jnp.int32) * VOCAB,
                    PERIOD // NUM_FIELDS)
    out = _sc_gather(ids_flat, offs, tables.reshape(NUM_FIELDS * VOCAB,
                                                    EMB_DIM))
    return out.reshape(BATCH, NUM_FIELDS * EMB_DIM)


# speedup vs baseline: 3.8747x; 3.1983x over previous
import functools

import jax
import jax.numpy as jnp
from jax import lax
from jax.experimental import pallas as pl
from jax.experimental.pallas import tpu as pltpu
from jax.experimental.pallas import tpu_sc as plsc

NUM_FIELDS = 26
VOCAB = 100000
EMB_DIM = 32
BATCH = 16384
NUM_WORKERS = 32


def _sc_gather_t(ids_t, tables_t):
    mesh = plsc.VectorSubcoreMesh(core_axis_name="c", subcore_axis_name="s")

    @functools.partial(
        pl.kernel,
        mesh=mesh,
        compiler_params=pltpu.CompilerParams(use_tc_tiling_on_sc=True,
                                             needs_layout_passes=False),
        out_type=jax.ShapeDtypeStruct((NUM_FIELDS * EMB_DIM, BATCH),
                                      jnp.float32),
        scratch_types=[
            pltpu.VMEM((VOCAB,), jnp.float32),
            pltpu.VMEM((BATCH // 2,), jnp.int32),
            pltpu.VMEM((BATCH,), jnp.float32),
            pltpu.SemaphoreType.DMA,
        ],
    )
    def k(ids_hbm, tab_hbm, out_hbm, row_v, ids_v, out_v, sem):
        d = lax.axis_index("s") * 2 + lax.axis_index("c")
        half = BATCH // 2

        def body(j, carry):
            pltpu.sync_copy(tab_hbm.at[j, d], row_v)
            for h in range(2):
                pltpu.sync_copy(ids_hbm.at[j, pl.ds(h * half, half)], ids_v)

                def gather16(i, c, h=h):
                    idx = ids_v[pl.ds(i * 16, 16)]
                    out_v[pl.ds(h * half + i * 16, 16)] = plsc.load_gather(
                        row_v, [idx])
                    return c

                lax.fori_loop(0, half // 16, gather16, 0)
            pltpu.sync_copy(out_v, out_hbm.at[j * EMB_DIM + d])
            return carry

        lax.fori_loop(0, NUM_FIELDS, body, 0)

    return k(ids_t, tables_t)


def kernel(x_cat_ids, tables):
    ids_t = x_cat_ids.T.astype(jnp.int32)
    tables_t = jnp.transpose(tables, (0, 2, 1))
    out_t = _sc_gather_t(ids_t, tables_t)
    return out_t.T


# speedup vs baseline: 6.5247x; 1.6839x over previous
import functools

import jax
import jax.numpy as jnp
from jax import lax
from jax.experimental import pallas as pl
from jax.experimental.pallas import tpu as pltpu
from jax.experimental.pallas import tpu_sc as plsc

NUM_FIELDS = 26
VOCAB = 100000
EMB_DIM = 32
BATCH = 16384
NUM_WORKERS = 32
CHUNK = 4096


def _sc_gather_t(ids_t, tables_t):
    mesh = plsc.VectorSubcoreMesh(core_axis_name="c", subcore_axis_name="s")

    @functools.partial(
        pl.kernel,
        mesh=mesh,
        compiler_params=pltpu.CompilerParams(use_tc_tiling_on_sc=True,
                                             needs_layout_passes=False),
        out_type=jax.ShapeDtypeStruct((NUM_FIELDS * EMB_DIM, BATCH),
                                      jnp.float32),
        scratch_types=[
            pltpu.VMEM((VOCAB,), jnp.float32),
            pltpu.VMEM((2, CHUNK), jnp.int32),
            pltpu.VMEM((BATCH,), jnp.float32),
            pltpu.SemaphoreType.DMA,
            pltpu.SemaphoreType.DMA,
            pltpu.SemaphoreType.DMA,
        ],
    )
    def k(ids_hbm, tab_hbm, out_hbm, row_v, ids_v, out_v, sem_row, sem_ids,
          sem_out):
        d = lax.axis_index("s") * 2 + lax.axis_index("c")
        nchunk = BATCH // CHUNK
        half = BATCH // 2

        def body(j, carry):
            row_cp = pltpu.async_copy(tab_hbm.at[j, d], row_v, sem_row)
            pltpu.async_copy(ids_hbm.at[j, pl.ds(0, CHUNK)],
                             ids_v.at[0], sem_ids).wait()
            row_cp.wait()
            for c in range(nchunk):
                if c + 1 < nchunk:
                    nxt = pltpu.async_copy(
                        ids_hbm.at[j, pl.ds((c + 1) * CHUNK, CHUNK)],
                        ids_v.at[(c + 1) % 2], sem_ids)

                @plsc.parallel_loop(0, CHUNK // 16, unroll=8)
                def gather16(i, c=c):
                    idx = ids_v[(c % 2), pl.ds(i * 16, 16)]
                    out_v[pl.ds(c * CHUNK + i * 16, 16)] = plsc.load_gather(
                        row_v, [idx])

                if c + 1 < nchunk:
                    nxt.wait()
                if (c + 1) * CHUNK == half:
                    cp0 = pltpu.async_copy(
                        out_v.at[pl.ds(0, half)],
                        out_hbm.at[j * EMB_DIM + d, pl.ds(0, half)], sem_out)
            cp1 = pltpu.async_copy(
                out_v.at[pl.ds(half, half)],
                out_hbm.at[j * EMB_DIM + d, pl.ds(half, half)], sem_out)
            cp0.wait()
            cp1.wait()
            return carry

        lax.fori_loop(0, NUM_FIELDS, body, 0)

    return k(ids_t, tables_t)


def kernel(x_cat_ids, tables):
    ids_t = x_cat_ids.T.astype(jnp.int32)
    tables_t = jnp.transpose(tables, (0, 2, 1))
    out_t = _sc_gather_t(ids_t, tables_t)
    return out_t.T


# speedup vs baseline: 7.4933x; 1.1484x over previous
import functools

import jax
import jax.numpy as jnp
from jax import lax
from jax.experimental import pallas as pl
from jax.experimental.pallas import tpu as pltpu
from jax.experimental.pallas import tpu_sc as plsc

NUM_FIELDS = 26
VOCAB = 100000
EMB_DIM = 32
BATCH = 16384
NUM_WORKERS = 32
CHUNK = 4096


def _sc_gather_t(ids_t, tables_t):
    mesh = plsc.VectorSubcoreMesh(core_axis_name="c", subcore_axis_name="s")

    @functools.partial(
        pl.kernel,
        mesh=mesh,
        compiler_params=pltpu.CompilerParams(use_tc_tiling_on_sc=True,
                                             needs_layout_passes=False),
        out_type=jax.ShapeDtypeStruct((NUM_FIELDS * EMB_DIM, BATCH),
                                      jnp.float32),
        scratch_types=[
            pltpu.VMEM((VOCAB,), jnp.float32),
            pltpu.VMEM((2, CHUNK), jnp.int32),
            pltpu.VMEM((2, CHUNK), jnp.float32),
            pltpu.VMEM_SHARED((2, BATCH), jnp.int32),
            pltpu.SemaphoreType.DMA,
            pltpu.SemaphoreType.DMA,
            pltpu.SemaphoreType.DMA,
        ],
    )
    def k(ids_hbm, tab_hbm, out_hbm, row_v, ids_v, out_v, ids_sh, sem_row,
          sem_ids, sem_out):
        d = lax.axis_index("s") * 2 + lax.axis_index("c")
        nchunk = BATCH // CHUNK

        @pl.when(lax.axis_index("s") == 0)
        def _stage0():
            pltpu.sync_copy(ids_hbm.at[0], ids_sh.at[0])

        plsc.subcore_barrier()

        def body(j, carry):
            jj = lax.rem(j, 2)
            row_cp = pltpu.async_copy(tab_hbm.at[j, d], row_v, sem_row)

            @pl.when(jnp.logical_and(lax.axis_index("s") == 0,
                                     j < NUM_FIELDS - 1))
            def _stage_next():
                pltpu.sync_copy(ids_hbm.at[j + 1], ids_sh.at[lax.rem(j + 1, 2)])

            pltpu.async_copy(ids_sh.at[jj, pl.ds(0, CHUNK)],
                             ids_v.at[0], sem_ids).wait()
            row_cp.wait()
            out_cps = [None] * nchunk
            for c in range(nchunk):
                if c + 1 < nchunk:
                    nxt = pltpu.async_copy(
                        ids_sh.at[jj, pl.ds((c + 1) * CHUNK, CHUNK)],
                        ids_v.at[(c + 1) % 2], sem_ids)
                if c >= 2:
                    out_cps[c - 2].wait()

                @plsc.parallel_loop(0, CHUNK // 16, unroll=8)
                def gather16(i, c=c):
                    idx = ids_v[(c % 2), pl.ds(i * 16, 16)]
                    out_v[(c % 2), pl.ds(i * 16, 16)] = plsc.load_gather(
                        row_v, [idx])

                out_cps[c] = pltpu.async_copy(
                    out_v.at[c % 2],
                    out_hbm.at[j * EMB_DIM + d, pl.ds(c * CHUNK, CHUNK)],
                    sem_out)
                if c + 1 < nchunk:
                    nxt.wait()
            out_cps[nchunk - 2].wait()
            out_cps[nchunk - 1].wait()
            plsc.subcore_barrier()
            return carry

        lax.fori_loop(0, NUM_FIELDS, body, 0)

    return k(ids_t, tables_t)


def kernel(x_cat_ids, tables):
    ids_t = x_cat_ids.T.astype(jnp.int32)
    tables_t = jnp.transpose(tables, (0, 2, 1))
    out_t = _sc_gather_t(ids_t, tables_t)
    return out_t.T


# speedup vs baseline: 7.5322x; 1.0052x over previous
import functools

import jax
import jax.numpy as jnp
from jax import lax
from jax.experimental import pallas as pl
from jax.experimental.pallas import tpu as pltpu
from jax.experimental.pallas import tpu_sc as plsc

NUM_FIELDS = 26
VOCAB = 100000
EMB_DIM = 32
BATCH = 16384
NUM_WORKERS = 32
CHUNK = 4096


def _sc_gather_t(ids_t, tables_t):
    mesh = plsc.VectorSubcoreMesh(core_axis_name="c", subcore_axis_name="s")

    @functools.partial(
        pl.kernel,
        mesh=mesh,
        compiler_params=pltpu.CompilerParams(use_tc_tiling_on_sc=True,
                                             needs_layout_passes=False),
        out_type=jax.ShapeDtypeStruct((NUM_FIELDS * EMB_DIM, BATCH),
                                      jnp.float32),
        scratch_types=[
            pltpu.VMEM((VOCAB,), jnp.float32),
            pltpu.VMEM((2, CHUNK), jnp.int32),
            pltpu.VMEM((2, 2 * CHUNK), jnp.float32),
            pltpu.VMEM_SHARED((2, BATCH), jnp.int32),
            pltpu.SemaphoreType.DMA,
            pltpu.SemaphoreType.DMA,
            pltpu.SemaphoreType.DMA,
        ],
    )
    def k(ids_hbm, tab_hbm, out_hbm, row_v, ids_v, out_v, ids_sh, sem_row,
          sem_ids, sem_out):
        d = lax.axis_index("s") * 2 + lax.axis_index("c")
        nchunk = BATCH // CHUNK

        @pl.when(lax.axis_index("s") == 0)
        def _stage0():
            pltpu.sync_copy(ids_hbm.at[0], ids_sh.at[0])

        plsc.subcore_barrier()

        def body(j, carry):
            jj = lax.rem(j, 2)
            row_cp = pltpu.async_copy(tab_hbm.at[j, d], row_v, sem_row)

            @pl.when(jnp.logical_and(lax.axis_index("s") == 0,
                                     j < NUM_FIELDS - 1))
            def _stage_next():
                pltpu.sync_copy(ids_hbm.at[j + 1], ids_sh.at[lax.rem(j + 1, 2)])

            pltpu.async_copy(ids_sh.at[jj, pl.ds(0, CHUNK)],
                             ids_v.at[0], sem_ids).wait()
            row_cp.wait()
            out_cps = [None, None]
            for c in range(nchunk):
                if c + 1 < nchunk:
                    nxt = pltpu.async_copy(
                        ids_sh.at[jj, pl.ds((c + 1) * CHUNK, CHUNK)],
                        ids_v.at[(c + 1) % 2], sem_ids)

                @plsc.parallel_loop(0, CHUNK // 16, unroll=16)
                def gather16(i, c=c):
                    idx = ids_v[(c % 2), pl.ds(i * 16, 16)]
                    out_v[(c // 2), pl.ds((c % 2) * CHUNK + i * 16, 16)] = (
                        plsc.load_gather(row_v, [idx]))

                if c % 2 == 1:
                    out_cps[c // 2] = pltpu.async_copy(
                        out_v.at[c // 2],
                        out_hbm.at[j * EMB_DIM + d,
                                   pl.ds((c // 2) * 2 * CHUNK, 2 * CHUNK)],
                        sem_out)
                if c + 1 < nchunk:
                    nxt.wait()
            out_cps[0].wait()
            out_cps[1].wait()
            plsc.subcore_barrier()
            return carry

        lax.fori_loop(0, NUM_FIELDS, body, 0)

    return k(ids_t, tables_t)


def kernel(x_cat_ids, tables):
    ids_t = x_cat_ids.T.astype(jnp.int32)
    tables_t = jnp.transpose(tables, (0, 2, 1))
    out_t = _sc_gather_t(ids_t, tables_t)
    return out_t.T


# speedup vs baseline: 7.8292x; 1.0394x over previous
import functools

import jax
import jax.numpy as jnp
from jax import lax
from jax.experimental import pallas as pl
from jax.experimental.pallas import tpu as pltpu
from jax.experimental.pallas import tpu_sc as plsc

NUM_FIELDS = 26
VOCAB = 100000
EMB_DIM = 32
BATCH = 16384
NUM_WORKERS = 32
CHUNK = 4096


def _sc_gather_t(ids_t, tables_t):
    mesh = plsc.VectorSubcoreMesh(core_axis_name="c", subcore_axis_name="s")

    @functools.partial(
        pl.kernel,
        mesh=mesh,
        compiler_params=pltpu.CompilerParams(use_tc_tiling_on_sc=True,
                                             needs_layout_passes=False),
        out_type=jax.ShapeDtypeStruct((NUM_FIELDS * EMB_DIM, BATCH),
                                      jnp.float32),
        scratch_types=[
            pltpu.VMEM((VOCAB,), jnp.float32),
            pltpu.VMEM((2, CHUNK), jnp.int32),
            pltpu.VMEM((2, 2 * CHUNK), jnp.float32),
            pltpu.VMEM_SHARED((2, BATCH), jnp.int32),
            pltpu.SemaphoreType.DMA,
            pltpu.SemaphoreType.DMA,
            pltpu.SemaphoreType.DMA,
        ],
    )
    def k(ids_hbm, tab_hbm, out_hbm, row_v, ids_v, out_v, ids_sh, sem_row,
          sem_ids, sem_out):
        d = lax.axis_index("s") * 2 + lax.axis_index("c")
        nchunk = BATCH // CHUNK

        @pl.when(lax.axis_index("s") == 0)
        def _stage0():
            pltpu.sync_copy(ids_hbm.at[0], ids_sh.at[0])

        pltpu.async_copy(tab_hbm.at[0, d], row_v, sem_row)
        plsc.subcore_barrier()

        def body(j, carry):
            jj = lax.rem(j, 2)

            @pl.when(jnp.logical_and(lax.axis_index("s") == 0,
                                     j < NUM_FIELDS - 1))
            def _stage_next():
                pltpu.sync_copy(ids_hbm.at[j + 1], ids_sh.at[lax.rem(j + 1, 2)])

            pltpu.async_copy(ids_sh.at[jj, pl.ds(0, CHUNK)],
                             ids_v.at[0], sem_ids).wait()
            pltpu.make_async_copy(tab_hbm.at[j, d], row_v, sem_row).wait()
            out_cps = [None, None]
            for c in range(nchunk):
                if c + 1 < nchunk:
                    nxt = pltpu.async_copy(
                        ids_sh.at[jj, pl.ds((c + 1) * CHUNK, CHUNK)],
                        ids_v.at[(c + 1) % 2], sem_ids)

                @plsc.parallel_loop(0, CHUNK // 16, unroll=16)
                def gather16(i, c=c):
                    idx = ids_v[(c % 2), pl.ds(i * 16, 16)]
                    out_v[(c // 2), pl.ds((c % 2) * CHUNK + i * 16, 16)] = (
                        plsc.load_gather(row_v, [idx]))

                if c % 2 == 1:
                    out_cps[c // 2] = pltpu.async_copy(
                        out_v.at[c // 2],
                        out_hbm.at[j * EMB_DIM + d,
                                   pl.ds((c // 2) * 2 * CHUNK, 2 * CHUNK)],
                        sem_out)
                if c + 1 < nchunk:
                    nxt.wait()
                if c == nchunk - 1:
                    @pl.when(j < NUM_FIELDS - 1)
                    def _next_row():
                        pltpu.async_copy(tab_hbm.at[j + 1, d], row_v, sem_row)

            out_cps[0].wait()
            out_cps[1].wait()
            plsc.subcore_barrier()
            return carry

        lax.fori_loop(0, NUM_FIELDS, body, 0)

    return k(ids_t, tables_t)


def kernel(x_cat_ids, tables):
    ids_t = x_cat_ids.T.astype(jnp.int32)
    tables_t = jnp.transpose(tables, (0, 2, 1))
    out_t = _sc_gather_t(ids_t, tables_t)
    return out_t.T


# speedup vs baseline: 7.8383x; 1.0012x over previous
import functools

import jax
import jax.numpy as jnp
from jax import lax
from jax.experimental import pallas as pl
from jax.experimental.pallas import tpu as pltpu
from jax.experimental.pallas import tpu_sc as plsc

NUM_FIELDS = 26
VOCAB = 100000
EMB_DIM = 32
BATCH = 16384
NUM_WORKERS = 32
CHUNK = 4096


def _sc_gather_t(ids_t, tables_t):
    mesh = plsc.VectorSubcoreMesh(core_axis_name="c", subcore_axis_name="s")

    @functools.partial(
        pl.kernel,
        mesh=mesh,
        compiler_params=pltpu.CompilerParams(use_tc_tiling_on_sc=True,
                                             needs_layout_passes=False),
        out_type=jax.ShapeDtypeStruct((NUM_FIELDS * EMB_DIM, BATCH),
                                      jnp.float32),
        scratch_types=[
            pltpu.VMEM((VOCAB,), jnp.float32),
            pltpu.VMEM((2, CHUNK), jnp.int32),
            pltpu.VMEM((2, 2 * CHUNK), jnp.float32),
            pltpu.VMEM_SHARED((2, BATCH), jnp.int32),
            pltpu.SemaphoreType.DMA,
            pltpu.SemaphoreType.DMA,
            pltpu.SemaphoreType.DMA,
        ],
    )
    def k(ids_hbm, tab_hbm, out_hbm, row_v, ids_v, out_v, ids_sh, sem_row,
          sem_ids, sem_out):
        d = lax.axis_index("c") * 16 + lax.axis_index("s")
        nchunk = BATCH // CHUNK

        @pl.when(lax.axis_index("s") == 0)
        def _stage0():
            pltpu.sync_copy(ids_hbm.at[0], ids_sh.at[0])

        pltpu.async_copy(tab_hbm.at[0, d], row_v, sem_row)
        plsc.subcore_barrier()

        def body(j, carry):
            jj = lax.rem(j, 2)

            @pl.when(jnp.logical_and(lax.axis_index("s") == 0,
                                     j < NUM_FIELDS - 1))
            def _stage_next():
                pltpu.sync_copy(ids_hbm.at[j + 1], ids_sh.at[lax.rem(j + 1, 2)])

            pltpu.async_copy(ids_sh.at[jj, pl.ds(0, CHUNK)],
                             ids_v.at[0], sem_ids).wait()
            pltpu.make_async_copy(tab_hbm.at[j, d], row_v, sem_row).wait()
            out_cps = [None, None]
            for c in range(nchunk):
                if c + 1 < nchunk:
                    nxt = pltpu.async_copy(
                        ids_sh.at[jj, pl.ds((c + 1) * CHUNK, CHUNK)],
                        ids_v.at[(c + 1) % 2], sem_ids)

                @plsc.parallel_loop(0, CHUNK // 16, unroll=16)
                def gather16(i, c=c):
                    idx = ids_v[(c % 2), pl.ds(i * 16, 16)]
                    out_v[(c // 2), pl.ds((c % 2) * CHUNK + i * 16, 16)] = (
                        plsc.load_gather(row_v, [idx]))

                if c % 2 == 1:
                    out_cps[c // 2] = pltpu.async_copy(
                        out_v.at[c // 2],
                        out_hbm.at[j * EMB_DIM + d,
                                   pl.ds((c // 2) * 2 * CHUNK, 2 * CHUNK)],
                        sem_out)
                if c + 1 < nchunk:
                    nxt.wait()
                if c == nchunk - 1:
                    @pl.when(j < NUM_FIELDS - 1)
                    def _next_row():
                        pltpu.async_copy(tab_hbm.at[j + 1, d], row_v, sem_row)

            plsc.subcore_barrier()
            out_cps[0].wait()
            out_cps[1].wait()
            return carry

        lax.fori_loop(0, NUM_FIELDS, body, 0)

    return k(ids_t, tables_t)


def kernel(x_cat_ids, tables):
    ids_t = x_cat_ids.T.astype(jnp.int32)
    tables_t = jnp.transpose(tables, (0, 2, 1))
    out_t = _sc_gather_t(ids_t, tables_t)
    return out_t.T
